# baseline (speedup 1.0000x reference)
import jax
import jax.numpy as jnp
from jax import lax
from jax.experimental import pallas as pl
from jax.experimental.pallas import tpu as pltpu
from jax.experimental.pallas import tpu_sc as plsc

B = 16384
L = 200
T = 100000
NC, NS = 2, 16
NW = NC * NS
ROWS_L = B // (NW // 2)
ROWS_R = B // NW
RC = 32
UNROLL = 8

_GDN = lax.GatherDimensionNumbers(offset_dims=(), collapsed_slice_dims=(0,),
                                  start_index_map=(0,))


def _reg_gather(table_vec, idx):
    return lax.gather(table_vec, idx[:, None], _GDN, (1,),
                      mode=lax.GatherScatterMode.PROMISE_IN_BOUNDS)


def _sc_body(sll, ill, ttl, sid, iid, cid, bid, tid,
             w_sll, w_ill, w_ttl, w_shop, w_item, w_cat, w_brand, w_tt,
             out_shop, out_item, out_rest,
             tbl, ib0, ib1, lacc, racc, sidx, svals, tt_s,
             sem_t, sem_a, sem_b, sem_g):
    c = lax.axis_index("c")
    s = lax.axis_index("s")
    wid = s * NC + c
    is_a = wid < (NW // 2)
    not_a = jnp.logical_not(is_a)
    wl = jnp.where(is_a, wid, wid - NW // 2)
    lbase = wl * ROWS_L
    rbase = wid * ROWS_R

    @pl.when(is_a)
    def _():
        pltpu.async_copy(w_sll, tbl, sem_t)

    @pl.when(not_a)
    def _():
        pltpu.async_copy(w_ill, tbl, sem_t)

    lane = lax.iota(jnp.int32, 16)

    pltpu.sync_copy(w_tt, tt_s)
    ttreg = tt_s[pl.ds(0, 16)]
    pltpu.sync_copy(tid.at[pl.ds(rbase, ROWS_R)], sidx)

    def tt_body(k, carry):
        iv = sidx[pl.ds(k * 16, 16)]
        v = _reg_gather(ttreg, iv)
        racc[pl.ds(k * 16, 16)] = jnp.where(iv != 0, v, 0.0)
        return carry

    lax.fori_loop(0, ROWS_R // 16, tt_body, 0)

    for idx_hbm, w_hbm in ((sid, w_shop), (iid, w_item), (cid, w_cat),
                           (bid, w_brand)):
        pltpu.sync_copy(idx_hbm.at[pl.ds(rbase, ROWS_R)], sidx)
        for j in range(ROWS_R // 128):
            pltpu.async_copy(w_hbm.at[sidx.at[pl.ds(j * 128, 128)]],
                             svals.at[pl.ds(j * 128, 128)], sem_g)
        for j in range(ROWS_R // 128):
            pltpu.make_async_copy(w_hbm.at[sidx.at[pl.ds(j * 128, 128)]],
                                  svals.at[pl.ds(j * 128, 128)], sem_g).wait()

        def s_body(k, carry):
            iv = sidx[pl.ds(k * 16, 16)]
            vv = svals[pl.ds(k * 16, 16)]
            racc[pl.ds(k * 16, 16)] = (racc[pl.ds(k * 16, 16)]
                                       + jnp.where(iv != 0, vv, 0.0))
            return carry

        lax.fori_loop(0, ROWS_R // 16, s_body, 0)

    def pooled(src2d, row0, nch, acc_ref, accumulate, table_ref=None,
               reg=None):
        def dma(ci, buf, sem):
            r = pl.multiple_of(row0 + ci * RC, RC)
            pltpu.async_copy(src2d.at[pl.ds(r, RC)], buf, sem)

        def drain(buf, sem):
            r0 = pl.multiple_of(row0, RC)
            pltpu.make_async_copy(src2d.at[pl.ds(r0, RC)], buf, sem).wait()

        def process(ci, buf):
            def grp_body(g, carry2):
                rowv = lane + g * 16

                def l_body(i, accs):
                    a0, a1 = accs
                    for u in range(UNROLL):
                        col = jnp.full((16,), i * UNROLL + u, jnp.int32)
                        iv = plsc.load_gather(buf, [rowv, col])
                        if table_ref is not None:
                            gv = plsc.load_gather(table_ref, [iv])
                        else:
                            gv = _reg_gather(reg, iv)
                        contrib = jnp.where(iv != 0, gv, 0.0)
                        if u % 2:
                            a1 = a1 + contrib
                        else:
                            a0 = a0 + contrib
                    return (a0, a1)

                z = jnp.zeros((16,), jnp.float32)
                a0, a1 = lax.fori_loop(0, L // UNROLL, l_body, (z, z))
                acc = a0 + a1
                o = ci * RC + g * 16
                if accumulate:
                    acc_ref[pl.ds(o, 16)] = acc_ref[pl.ds(o, 16)] + acc
                else:
                    acc_ref[pl.ds(o, 16)] = acc
                return carry2

            lax.fori_loop(0, RC // 16, grp_body, 0)

        dma(0, ib0, sem_a)
        dma(1, ib1, sem_b)

        def pair_body(p, carry):
            ci0 = 2 * p
            drain(ib0, sem_a)
            process(ci0, ib0)

            @pl.when(ci0 + 2 < nch)
            def _():
                dma(ci0 + 2, ib0, sem_a)

            drain(ib1, sem_b)
            process(ci0 + 1, ib1)

            @pl.when(ci0 + 3 < nch)
            def _():
                dma(ci0 + 3, ib1, sem_b)

            return carry

        lax.fori_loop(0, nch // 2, pair_body, 0)

    pltpu.sync_copy(w_ttl, tt_s)
    tlreg = tt_s[pl.ds(0, 16)]
    pooled(ttl, rbase, ROWS_R // RC, racc, True, reg=tlreg)

    pltpu.make_async_copy(w_sll, tbl, sem_t).wait()

    @pl.when(is_a)
    def _():
        pooled(sll, lbase, ROWS_L // RC, lacc, False, table_ref=tbl)

    @pl.when(not_a)
    def _():
        pooled(ill, lbase, ROWS_L // RC, lacc, False, table_ref=tbl)

    @pl.when(is_a)
    def _():
        pltpu.sync_copy(lacc, out_shop.at[pl.ds(lbase, ROWS_L)])

    @pl.when(not_a)
    def _():
        pltpu.sync_copy(lacc, out_item.at[pl.ds(lbase, ROWS_L)])

    pltpu.sync_copy(racc, out_rest.at[pl.ds(rbase, ROWS_R)])


_sc_call = pl.kernel(
    _sc_body,
    out_type=(jax.ShapeDtypeStruct((B,), jnp.float32),
              jax.ShapeDtypeStruct((B,), jnp.float32),
              jax.ShapeDtypeStruct((B,), jnp.float32)),
    mesh=plsc.VectorSubcoreMesh(core_axis_name="c", subcore_axis_name="s"),
    compiler_params=pltpu.CompilerParams(needs_layout_passes=False),
    scratch_types=[
        pltpu.VMEM((T,), jnp.float32),
        pltpu.VMEM((RC, L), jnp.int32),
        pltpu.VMEM((RC, L), jnp.int32),
        pltpu.VMEM((ROWS_L,), jnp.float32),
        pltpu.VMEM((ROWS_R,), jnp.float32),
        pltpu.VMEM((ROWS_R,), jnp.int32),
        pltpu.VMEM((ROWS_R,), jnp.float32),
        pltpu.VMEM((128,), jnp.float32),
        pltpu.SemaphoreType.DMA,
        pltpu.SemaphoreType.DMA,
        pltpu.SemaphoreType.DMA,
        pltpu.SemaphoreType.DMA,
    ],
)

BLK = 2048


def _tc_body(price_ref, hlist_ref, r7, r30, r90, hr, ps, pi_, pr, out_ref):
    srow = (jnp.sum(price_ref[...], axis=1, keepdims=True)
            + jnp.sum(hlist_ref[...], axis=1, keepdims=True))
    out_ref[...] = (srow + r7[...] + r30[...] + r90[...] + hr[...]
                    + ps[...] + pi_[...] + pr[...])


def _tc_call(price_list, hours_list, rank_7, rank_30, rank_90, hours,
             p_shop, p_item, p_rest):
    col = pl.BlockSpec((BLK, 1), lambda i: (i, 0))
    mat = pl.BlockSpec((BLK, L), lambda i: (i, 0))
    return pl.pallas_call(
        _tc_body,
        grid=(B // BLK,),
        in_specs=[mat, mat, col, col, col, col, col, col, col],
        out_specs=col,
        out_shape=jax.ShapeDtypeStruct((B, 1), jnp.float32),
    )(price_list, hours_list, rank_7, rank_30, rank_90, hours,
      p_shop, p_item, p_rest)


def kernel(shop_id, item_id, category_1_id, brand_id, time_type,
           shop_id_list, item_id_list, time_type_list,
           rank_7, rank_30, rank_90, hours, price_list, hours_list,
           W_shop_id, W_item_id, W_category_1_id, W_brand_id, W_time_type,
           W_shop_id_list, W_item_id_list, W_time_type_list):
    w_tt = jnp.pad(W_time_type.reshape(-1), (0, 128 - W_time_type.shape[0]))
    w_ttl = jnp.pad(W_time_type_list.reshape(-1),
                    (0, 128 - W_time_type_list.shape[0]))
    p_shop, p_item, p_rest = _sc_call(
        shop_id_list, item_id_list, time_type_list,
        shop_id.astype(jnp.int32), item_id.astype(jnp.int32),
        category_1_id.astype(jnp.int32), brand_id.astype(jnp.int32),
        time_type.astype(jnp.int32),
        W_shop_id_list.reshape(-1), W_item_id_list.reshape(-1), w_ttl,
        W_shop_id.reshape(-1), W_item_id.reshape(-1),
        W_category_1_id.reshape(-1), W_brand_id.reshape(-1), w_tt)
    return _tc_call(price_list, hours_list, rank_7, rank_30, rank_90, hours,
                    p_shop.reshape(B, 1), p_item.reshape(B, 1),
                    p_rest.reshape(B, 1))


# speedup vs baseline: 2.7072x; 2.7072x over previous
import jax
import jax.numpy as jnp
from jax import lax
from jax.experimental import pallas as pl
from jax.experimental.pallas import tpu as pltpu
from jax.experimental.pallas import tpu_sc as plsc

B = 16384
L = 200
T = 100000
NC, NS = 2, 16
NW = NC * NS
ROWS_L = B // (NW // 2)
ROWS_R = B // NW
PC = 8
NCH = L // PC

_GDN = lax.GatherDimensionNumbers(offset_dims=(), collapsed_slice_dims=(0,),
                                  start_index_map=(0,))


def _reg_gather(table_vec, idx):
    return lax.gather(table_vec, idx[:, None], _GDN, (1,),
                      mode=lax.GatherScatterMode.PROMISE_IN_BOUNDS)


def _sc_body(sll_t, ill_t, ttl_t, sid, iid, cid, bid, tid,
             w_sll, w_ill, w_ttl, w_shop, w_item, w_cat, w_brand, w_tt,
             out_shop, out_item, out_rest,
             tbl, lb0, lb1, tb0, tb1, lacc, racc, sidx, svals, tt_s,
             sem_t, sem_a, sem_b, sem_c, sem_d, sem_g):
    c = lax.axis_index("c")
    s = lax.axis_index("s")
    wid = s * NC + c
    is_a = wid < (NW // 2)
    not_a = jnp.logical_not(is_a)
    wl = jnp.where(is_a, wid, wid - NW // 2)
    lbase = wl * ROWS_L
    rbase = wid * ROWS_R

    @pl.when(is_a)
    def _():
        pltpu.async_copy(w_sll, tbl, sem_t)

    @pl.when(not_a)
    def _():
        pltpu.async_copy(w_ill, tbl, sem_t)

    pltpu.sync_copy(w_tt, tt_s)
    ttreg = tt_s[pl.ds(0, 16)]
    pltpu.sync_copy(tid.at[pl.ds(rbase, ROWS_R)], sidx)

    def tt_body(k, carry):
        iv = sidx[pl.ds(k * 16, 16)]
        v = _reg_gather(ttreg, iv)
        racc[pl.ds(k * 16, 16)] = jnp.where(iv != 0, v, 0.0)
        return carry

    lax.fori_loop(0, ROWS_R // 16, tt_body, 0)

    for idx_hbm, w_hbm in ((sid, w_shop), (iid, w_item), (cid, w_cat),
                           (bid, w_brand)):
        pltpu.sync_copy(idx_hbm.at[pl.ds(rbase, ROWS_R)], sidx)
        for j in range(ROWS_R // 128):
            pltpu.async_copy(w_hbm.at[sidx.at[pl.ds(j * 128, 128)]],
                             svals.at[pl.ds(j * 128, 128)], sem_g)
        for j in range(ROWS_R // 128):
            pltpu.make_async_copy(w_hbm.at[sidx.at[pl.ds(j * 128, 128)]],
                                  svals.at[pl.ds(j * 128, 128)], sem_g).wait()

        def s_body(k, carry):
            iv = sidx[pl.ds(k * 16, 16)]
            vv = svals[pl.ds(k * 16, 16)]
            racc[pl.ds(k * 16, 16)] = (racc[pl.ds(k * 16, 16)]
                                       + jnp.where(iv != 0, vv, 0.0))
            return carry

        lax.fori_loop(0, ROWS_R // 16, s_body, 0)

    def pooled(src_t, bbase, width, acc_ref, b0, b1, s0, s1,
               table_ref=None, reg=None):
        def dma(ci, buf, sem):
            p0 = pl.multiple_of(ci * PC, PC)
            bb = pl.multiple_of(bbase, 128)
            pltpu.async_copy(src_t.at[pl.ds(p0, PC), pl.ds(bb, width)],
                             buf, sem)

        def drain(buf, sem):
            bb = pl.multiple_of(bbase, 128)
            pltpu.make_async_copy(src_t.at[pl.ds(0, PC), pl.ds(bb, width)],
                                  buf, sem).wait()

        def process(buf):
            def g_body(g, carry):
                o = g * 16
                acc = acc_ref[pl.ds(o, 16)]
                for l in range(PC):
                    iv = buf[l, pl.ds(o, 16)]
                    if table_ref is not None:
                        gv = plsc.load_gather(table_ref, [iv])
                    else:
                        gv = _reg_gather(reg, iv)
                    acc = acc + jnp.where(iv != 0, gv, 0.0)
                acc_ref[pl.ds(o, 16)] = acc
                return carry

            lax.fori_loop(0, width // 16, g_body, 0)

        dma(0, b0, s0)
        dma(1, b1, s1)

        def pair_body(p, carry):
            ci0 = 2 * p
            drain(b0, s0)
            process(b0)

            @pl.when(ci0 + 2 < NCH)
            def _():
                dma(ci0 + 2, b0, s0)

            drain(b1, s1)
            process(b1)

            @pl.when(ci0 + 3 < NCH)
            def _():
                dma(ci0 + 3, b1, s1)

            return carry

        lax.fori_loop(0, NCH // 2, pair_body, 0)
        drain(b0, s0)
        process(b0)

    pltpu.sync_copy(w_ttl, tt_s)
    tlreg = tt_s[pl.ds(0, 16)]
    pooled(ttl_t, rbase, ROWS_R, racc, tb0, tb1, sem_c, sem_d, reg=tlreg)

    zv = jnp.zeros((16,), jnp.float32)

    def z_body(g, carry):
        lacc[pl.ds(g * 16, 16)] = zv
        return carry

    lax.fori_loop(0, ROWS_L // 16, z_body, 0)
    pltpu.make_async_copy(w_sll, tbl, sem_t).wait()

    @pl.when(is_a)
    def _():
        pooled(sll_t, lbase, ROWS_L, lacc, lb0, lb1, sem_a, sem_b,
               table_ref=tbl)

    @pl.when(not_a)
    def _():
        pooled(ill_t, lbase, ROWS_L, lacc, lb0, lb1, sem_a, sem_b,
               table_ref=tbl)

    @pl.when(is_a)
    def _():
        pltpu.sync_copy(lacc, out_shop.at[pl.ds(lbase, ROWS_L)])

    @pl.when(not_a)
    def _():
        pltpu.sync_copy(lacc, out_item.at[pl.ds(lbase, ROWS_L)])

    pltpu.sync_copy(racc, out_rest.at[pl.ds(rbase, ROWS_R)])


_sc_call = pl.kernel(
    _sc_body,
    out_type=(jax.ShapeDtypeStruct((B,), jnp.float32),
              jax.ShapeDtypeStruct((B,), jnp.float32),
              jax.ShapeDtypeStruct((B,), jnp.float32)),
    mesh=plsc.VectorSubcoreMesh(core_axis_name="c", subcore_axis_name="s"),
    compiler_params=pltpu.CompilerParams(needs_layout_passes=False),
    scratch_types=[
        pltpu.VMEM((T,), jnp.float32),
        pltpu.VMEM((PC, ROWS_L), jnp.int32),
        pltpu.VMEM((PC, ROWS_L), jnp.int32),
        pltpu.VMEM((PC, ROWS_R), jnp.int32),
        pltpu.VMEM((PC, ROWS_R), jnp.int32),
        pltpu.VMEM((ROWS_L,), jnp.float32),
        pltpu.VMEM((ROWS_R,), jnp.float32),
        pltpu.VMEM((ROWS_R,), jnp.int32),
        pltpu.VMEM((ROWS_R,), jnp.float32),
        pltpu.VMEM((128,), jnp.float32),
        pltpu.SemaphoreType.DMA,
        pltpu.SemaphoreType.DMA,
        pltpu.SemaphoreType.DMA,
        pltpu.SemaphoreType.DMA,
        pltpu.SemaphoreType.DMA,
        pltpu.SemaphoreType.DMA,
    ],
)

BLKB = 2048


def _tc_body(price_t, hlist_t, r7, r30, r90, hr, ps, pi_, pr, out_ref):
    srow = jnp.sum(price_t[...], axis=0) + jnp.sum(hlist_t[...], axis=0)
    out_ref[...] = (srow + r7[...] + r30[...] + r90[...] + hr[...]
                    + ps[...] + pi_[...] + pr[...])


def _tc_call(price_t, hours_t, rank_7, rank_30, rank_90, hours,
             p_shop, p_item, p_rest):
    vec = pl.BlockSpec((BLKB,), lambda i: (i,))
    mat = pl.BlockSpec((L, BLKB), lambda i: (0, i))
    return pl.pallas_call(
        _tc_body,
        grid=(B // BLKB,),
        in_specs=[mat, mat, vec, vec, vec, vec, vec, vec, vec],
        out_specs=vec,
        out_shape=jax.ShapeDtypeStruct((B,), jnp.float32),
    )(price_t, hours_t, rank_7, rank_30, rank_90, hours,
      p_shop, p_item, p_rest)


def kernel(shop_id, item_id, category_1_id, brand_id, time_type,
           shop_id_list, item_id_list, time_type_list,
           rank_7, rank_30, rank_90, hours, price_list, hours_list,
           W_shop_id, W_item_id, W_category_1_id, W_brand_id, W_time_type,
           W_shop_id_list, W_item_id_list, W_time_type_list):
    w_tt = jnp.pad(W_time_type.reshape(-1), (0, 128 - W_time_type.shape[0]))
    w_ttl = jnp.pad(W_time_type_list.reshape(-1),
                    (0, 128 - W_time_type_list.shape[0]))
    p_shop, p_item, p_rest = _sc_call(
        shop_id_list.T, item_id_list.T, time_type_list.T,
        shop_id.astype(jnp.int32), item_id.astype(jnp.int32),
        category_1_id.astype(jnp.int32), brand_id.astype(jnp.int32),
        time_type.astype(jnp.int32),
        W_shop_id_list.reshape(-1), W_item_id_list.reshape(-1), w_ttl,
        W_shop_id.reshape(-1), W_item_id.reshape(-1),
        W_category_1_id.reshape(-1), W_brand_id.reshape(-1), w_tt)
    out = _tc_call(price_list.T, hours_list.T,
                   rank_7.reshape(-1), rank_30.reshape(-1),
                   rank_90.reshape(-1), hours.reshape(-1),
                   p_shop, p_item, p_rest)
    return out.reshape(B, 1)
